# baseline (speedup 1.0000x reference)
import jax
import jax.numpy as jnp
from jax import lax
from jax.experimental import pallas as pl
from jax.experimental.pallas import tpu as pltpu
from jax.experimental.pallas import tpu_sc as plsc

BATCH = 16384
NROWS = 1000001
DIM = 32
NW = 32
BPW = BATCH // NW
CHUNK = 128
NCHUNK = BPW // CHUNK
GROUPS = BPW // 16
TCOLS = 512
TSTEPS = (NROWS + TCOLS - 1) // TCOLS
PROWS = TSTEPS * TCOLS // 4


def _prep_body(t_ref, out_ref):
    x = t_ref[...]
    r = lax.broadcasted_iota(jnp.int32, (TCOLS, TCOLS), 0)
    c = lax.broadcasted_iota(jnp.int32, (TCOLS, TCOLS), 1)
    eye = (r == c).astype(jnp.float32)
    y = lax.dot_general(eye, x, (((1,), (1,)), ((), ())),
                        preferred_element_type=jnp.float32)
    for t in range(4):
        out_ref[:, pl.ds(32 * t, 32)] = y[128 * t:128 * (t + 1), :]


_prep_call = pl.pallas_call(
    _prep_body,
    grid=(TSTEPS,),
    in_specs=[pl.BlockSpec((DIM, TCOLS), lambda i: (0, i))],
    out_specs=pl.BlockSpec((TCOLS // 4, 128), lambda i: (i, 0)),
    out_shape=jax.ShapeDtypeStruct((PROWS, 128), jnp.float32),
)


def _sc_body(user_hbm, item_hbm, upk_hbm, ipk_hbm, ubias_hbm, ibias_hbm,
             logits_hbm, reg_hbm,
             idx_u, idx_i, q_u, q_i, rows_u, rows_i, bias_u, bias_i,
             logits_v, sq_v, sem):
    wid = lax.axis_index("s") * 2 + lax.axis_index("c")
    base = wid * BPW

    pltpu.sync_copy(user_hbm.at[wid], idx_u)
    pltpu.sync_copy(item_hbm.at[wid], idx_i)
    for j in range(NCHUNK):
        for k in range(CHUNK // 16):
            sl = pl.ds(k * 16, 16)
            ru = idx_u[j, sl]
            ri = idx_i[j, sl]
            q_u[j, sl] = ((ru >> 9) << 7) + (ru & 127)
            q_i[j, sl] = ((ri >> 9) << 7) + (ri & 127)

    lane = lax.iota(jnp.int32, 16)
    sq = jnp.zeros((16,), jnp.float32)

    bias_copies = []
    for j in range(NCHUNK):
        sl = pl.ds(j * CHUNK, CHUNK)
        bias_copies.append(
            pltpu.async_copy(ubias_hbm.at[idx_u.at[j]], bias_u.at[sl], sem))
        bias_copies.append(
            pltpu.async_copy(ibias_hbm.at[idx_i.at[j]], bias_i.at[sl], sem))

    for p in range(2):
        copies = []
        for jj in range(2):
            j = p * 2 + jj
            sl = pl.ds(jj * CHUNK, CHUNK)
            copies.append(pltpu.async_copy(upk_hbm.at[q_u.at[j]], rows_u.at[sl], sem))
            copies.append(pltpu.async_copy(ipk_hbm.at[q_i.at[j]], rows_i.at[sl], sem))
        if p == 0:
            copies.extend(bias_copies)
        for c in copies:
            c.wait()

        def group(gl, sq_acc):
            g = p * 16 + gl
            coli = gl * 16 + lane
            jv = jnp.full((16,), 0, jnp.int32) + (g >> 3)
            kv = (g & 7) * 16 + lane
            riv_u = plsc.load_gather(idx_u, [jv, kv])
            riv_i = plsc.load_gather(idx_i, [jv, kv])
            offu = ((riv_u >> 7) & 3) * DIM
            offi = ((riv_i >> 7) & 3) * DIM
            dot = jnp.zeros((16,), jnp.float32)
            for d in range(DIM):
                u = plsc.load_gather(rows_u, [coli, offu + d])
                v = plsc.load_gather(rows_i, [coli, offi + d])
                dot = dot + u * v
                sq_acc = sq_acc + u * u + v * v
            ub = plsc.load_gather(bias_u, [g * 16 + lane])
            ib = plsc.load_gather(bias_i, [g * 16 + lane])
            logits_v[pl.ds(g * 16, 16)] = dot + ub + ib
            return sq_acc

        sq = lax.fori_loop(0, GROUPS // 2, group, sq)
    sq_v[...] = sq
    pltpu.sync_copy(logits_v, logits_hbm.at[pl.ds(base, BPW)])
    pltpu.sync_copy(sq_v, reg_hbm.at[wid])


_sc_call = pl.kernel(
    _sc_body,
    out_type=[
        jax.ShapeDtypeStruct((BATCH,), jnp.float32),
        jax.ShapeDtypeStruct((NW, 16), jnp.float32),
    ],
    mesh=plsc.VectorSubcoreMesh(core_axis_name="c", subcore_axis_name="s"),
    compiler_params=pltpu.CompilerParams(needs_layout_passes=False,
                                         use_tc_tiling_on_sc=False),
    scratch_types=[
        pltpu.VMEM((NCHUNK, CHUNK), jnp.int32),
        pltpu.VMEM((NCHUNK, CHUNK), jnp.int32),
        pltpu.VMEM((NCHUNK, CHUNK), jnp.int32),
        pltpu.VMEM((NCHUNK, CHUNK), jnp.int32),
        pltpu.VMEM((BPW // 2, 128), jnp.float32),
        pltpu.VMEM((BPW // 2, 128), jnp.float32),
        pltpu.VMEM((BPW,), jnp.float32),
        pltpu.VMEM((BPW,), jnp.float32),
        pltpu.VMEM((BPW,), jnp.float32),
        pltpu.VMEM((16,), jnp.float32),
        pltpu.SemaphoreType.DMA,
    ],
)


def _tc_body(logits_ref, ratings_ref, reg_ref, pre_ref, loss_ref, regloss_ref):
    pre = jax.nn.sigmoid(logits_ref[...])
    pre_ref[...] = pre
    err = pre - ratings_ref[...]
    loss_ref[0, 0] = jnp.sum(err * err) / float(BATCH)
    regloss_ref[0, 0] = 0.5 * jnp.sum(reg_ref[...]) / float(BATCH)


_tc_call = pl.pallas_call(
    _tc_body,
    out_shape=[
        jax.ShapeDtypeStruct((128, 128), jnp.float32),
        jax.ShapeDtypeStruct((1, 1), jnp.float32),
        jax.ShapeDtypeStruct((1, 1), jnp.float32),
    ],
    out_specs=[
        pl.BlockSpec(memory_space=pltpu.VMEM),
        pl.BlockSpec(memory_space=pltpu.SMEM),
        pl.BlockSpec(memory_space=pltpu.SMEM),
    ],
)


def kernel(user, item, ratings, user_emb_table, item_emb_table,
           user_bias_table, item_bias_table):
    user_r = user.astype(jnp.int32).reshape(NW, NCHUNK, CHUNK)
    item_r = item.astype(jnp.int32).reshape(NW, NCHUNK, CHUNK)
    upk = _prep_call(user_emb_table.T)
    ipk = _prep_call(item_emb_table.T)
    ub1 = user_bias_table.sum(axis=1)
    ib1 = item_bias_table.sum(axis=1)
    logits, regpart = _sc_call(user_r, item_r, upk, ipk, ub1, ib1)
    pre2d, loss, regloss = _tc_call(logits.reshape(128, 128),
                                    ratings.reshape(128, 128),
                                    regpart.reshape(4, 128))
    return (loss[0, 0], regloss[0, 0], pre2d.reshape(BATCH))


# speedup vs baseline: 2.8891x; 2.8891x over previous
import jax
import jax.numpy as jnp
from jax import lax
from jax.experimental import pallas as pl
from jax.experimental.pallas import tpu as pltpu
from jax.experimental.pallas import tpu_sc as plsc

BATCH = 16384
NROWS = 1000001
DIM = 32
NW = 32
BPW = BATCH // NW
CHUNK = 128
NCHUNK = BPW // CHUNK
GROUPS = BPW // 16


def _sc_body(user_hbm, item_hbm, uemb_hbm, iemb_hbm, ubias_hbm, ibias_hbm,
             logits_hbm, reg_hbm,
             idx_u, idx_i, rows_u, rows_i, bias_u, bias_i, logits_v, sq_v,
             sem):
    wid = lax.axis_index("s") * 2 + lax.axis_index("c")
    base = wid * BPW

    pltpu.sync_copy(user_hbm.at[wid], idx_u)
    pltpu.sync_copy(item_hbm.at[wid], idx_i)
    copies = []
    for j in range(NCHUNK):
        sl = pl.ds(j * CHUNK, CHUNK)
        copies.append(pltpu.async_copy(uemb_hbm.at[idx_u.at[j]], rows_u.at[sl], sem))
        copies.append(pltpu.async_copy(iemb_hbm.at[idx_i.at[j]], rows_i.at[sl], sem))
        copies.append(pltpu.async_copy(ubias_hbm.at[idx_u.at[j]], bias_u.at[sl], sem))
        copies.append(pltpu.async_copy(ibias_hbm.at[idx_i.at[j]], bias_i.at[sl], sem))
    for c in copies:
        c.wait()

    lane = lax.iota(jnp.int32, 16)

    def group(g, sq_acc):
        rowi = g * 16 + lane
        dot = jnp.zeros((16,), jnp.float32)
        for d in range(DIM):
            cd = jnp.full((16,), d, jnp.int32)
            u = plsc.load_gather(rows_u, [rowi, cd])
            v = plsc.load_gather(rows_i, [rowi, cd])
            dot = dot + u * v
            sq_acc = sq_acc + u * u + v * v
        ub = plsc.load_gather(bias_u, [rowi])
        ib = plsc.load_gather(bias_i, [rowi])
        logits_v[pl.ds(g * 16, 16)] = dot + ub + ib
        return sq_acc

    sq = lax.fori_loop(0, GROUPS, group, jnp.zeros((16,), jnp.float32))
    sq_v[...] = sq
    pltpu.sync_copy(logits_v, logits_hbm.at[pl.ds(base, BPW)])
    pltpu.sync_copy(sq_v, reg_hbm.at[wid])


_sc_call = pl.kernel(
    _sc_body,
    out_type=[
        jax.ShapeDtypeStruct((BATCH,), jnp.float32),
        jax.ShapeDtypeStruct((NW, 16), jnp.float32),
    ],
    mesh=plsc.VectorSubcoreMesh(core_axis_name="c", subcore_axis_name="s"),
    compiler_params=pltpu.CompilerParams(needs_layout_passes=False,
                                         use_tc_tiling_on_sc=False),
    scratch_types=[
        pltpu.VMEM((NCHUNK, CHUNK), jnp.int32),
        pltpu.VMEM((NCHUNK, CHUNK), jnp.int32),
        pltpu.VMEM((BPW, DIM), jnp.float32),
        pltpu.VMEM((BPW, DIM), jnp.float32),
        pltpu.VMEM((BPW,), jnp.float32),
        pltpu.VMEM((BPW,), jnp.float32),
        pltpu.VMEM((BPW,), jnp.float32),
        pltpu.VMEM((16,), jnp.float32),
        pltpu.SemaphoreType.DMA,
    ],
)


def _tc_body(logits_ref, ratings_ref, reg_ref, pre_ref, loss_ref, regloss_ref):
    pre = jax.nn.sigmoid(logits_ref[...])
    pre_ref[...] = pre
    err = pre - ratings_ref[...]
    loss_ref[0, 0] = jnp.sum(err * err) / float(BATCH)
    regloss_ref[0, 0] = 0.5 * jnp.sum(reg_ref[...]) / float(BATCH)


_tc_call = pl.pallas_call(
    _tc_body,
    out_shape=[
        jax.ShapeDtypeStruct((128, 128), jnp.float32),
        jax.ShapeDtypeStruct((1, 1), jnp.float32),
        jax.ShapeDtypeStruct((1, 1), jnp.float32),
    ],
    out_specs=[
        pl.BlockSpec(memory_space=pltpu.VMEM),
        pl.BlockSpec(memory_space=pltpu.SMEM),
        pl.BlockSpec(memory_space=pltpu.SMEM),
    ],
)


def kernel(user, item, ratings, user_emb_table, item_emb_table,
           user_bias_table, item_bias_table):
    user_r = user.astype(jnp.int32).reshape(NW, NCHUNK, CHUNK)
    item_r = item.astype(jnp.int32).reshape(NW, NCHUNK, CHUNK)
    ub1 = user_bias_table.sum(axis=1)
    ib1 = item_bias_table.sum(axis=1)
    logits, regpart = _sc_call(user_r, item_r, user_emb_table, item_emb_table,
                               ub1, ib1)
    pre2d, loss, regloss = _tc_call(logits.reshape(128, 128),
                                    ratings.reshape(128, 128),
                                    regpart.reshape(4, 128))
    return (loss[0, 0], regloss[0, 0], pre2d.reshape(BATCH))
